# baseline (speedup 1.0000x reference)
import functools

import jax
import jax.numpy as jnp
from jax import lax
from jax.experimental import pallas as pl
from jax.experimental.pallas import tpu as pltpu
from jax.experimental.pallas import tpu_sc as plsc

_NC = 2
_NS = 16
_NW = _NC * _NS
_L = 16
_N_ACC = 4


def _make_sc_partials(B, H, W):
    HW = H * W
    assert B % (2 * _NW) == 0 and HW % (_L * _N_ACC) == 0
    bpw = B // _NW
    n_steps = HW // _L
    mesh = plsc.VectorSubcoreMesh(core_axis_name="c", subcore_axis_name="s")

    @functools.partial(
        pl.kernel,
        mesh=mesh,
        compiler_params=pltpu.CompilerParams(needs_layout_passes=False),
        out_type=[
            jax.ShapeDtypeStruct((_NW, _L), jnp.float32),
            jax.ShapeDtypeStruct((_NW, _L), jnp.float32),
        ],
        scratch_types=[
            pltpu.VMEM((HW,), jnp.float32),
            pltpu.VMEM((HW,), jnp.float32),
            pltpu.VMEM((HW,), jnp.float32),
            pltpu.VMEM((HW,), jnp.float32),
            pltpu.VMEM((HW,), jnp.float32),
            pltpu.VMEM((HW,), jnp.float32),
            pltpu.VMEM((HW,), jnp.float32),
            pltpu.VMEM((HW,), jnp.float32),
            pltpu.VMEM((HW,), jnp.float32),
            pltpu.VMEM((HW,), jnp.float32),
            pltpu.VMEM((_L,), jnp.float32),
            pltpu.VMEM((_L,), jnp.float32),
            pltpu.SemaphoreType.DMA,
            pltpu.SemaphoreType.DMA,
        ],
    )
    def sc_kernel(curr_hbm, mask_hbm, roi_hbm, loss_out, cnt_out,
                  y_v0, y_v1, x_v0, x_v1, t_v0, t_v1,
                  curr_v0, curr_v1, mask_v0, mask_v1,
                  loss_v, cnt_v, sem0, sem1):
        wid = lax.axis_index("s") * _NC + lax.axis_index("c")
        base = wid * bpw
        bufs = ((y_v0, x_v0, t_v0, curr_v0, mask_v0, sem0),
                (y_v1, x_v1, t_v1, curr_v1, mask_v1, sem1))

        def fire(b, k):
            y_v, x_v, t_v, curr_v, mask_v, sem = bufs[k]
            pltpu.make_async_copy(curr_hbm.at[b], curr_v, sem).start()
            pltpu.make_async_copy(mask_hbm.at[b], mask_v, sem).start()
            pltpu.make_async_copy(roi_hbm.at[0, b], y_v, sem).start()
            pltpu.make_async_copy(roi_hbm.at[1, b], x_v, sem).start()
            pltpu.make_async_copy(roi_hbm.at[2, b], t_v, sem).start()

        def drain(b, k):
            y_v, x_v, t_v, curr_v, mask_v, sem = bufs[k]
            pltpu.make_async_copy(curr_hbm.at[b], curr_v, sem).wait()
            pltpu.make_async_copy(mask_hbm.at[b], mask_v, sem).wait()
            pltpu.make_async_copy(roi_hbm.at[0, b], y_v, sem).wait()
            pltpu.make_async_copy(roi_hbm.at[1, b], x_v, sem).wait()
            pltpu.make_async_copy(roi_hbm.at[2, b], t_v, sem).wait()

        def compute(k, accs):
            y_v, x_v, t_v, curr_v, mask_v, _ = bufs[k]

            def one(j):
                sl = pl.ds(j * _L, _L)
                y = y_v[sl].astype(jnp.int32)
                x = x_v[sl].astype(jnp.int32)
                t = t_v[sl]
                valid = (lax.bitcast_convert_type(y, jnp.uint32) < H) & (
                    lax.bitcast_convert_type(x, jnp.uint32) < W)
                flat_u = lax.bitcast_convert_type(y * W + x, jnp.uint32)
                flat = lax.bitcast_convert_type(
                    jnp.minimum(flat_u, jnp.uint32(HW - 1)), jnp.int32)
                c = plsc.load_gather(curr_v, [flat])
                m = plsc.load_gather(mask_v, [flat])
                d = c - t
                ad = jnp.abs(d)
                loss = jnp.where(ad < 1.0, 0.5 * d * d, ad - 0.5)
                w = jnp.where(valid, m, 0.0)
                return loss * w, w

            def group(g, accs2):
                out = []
                for q in range(_N_ACC):
                    lacc, cacc = accs2[q]
                    lw, w = one(g * _N_ACC + q)
                    out.append((lacc + lw, cacc + w))
                return tuple(out)

            return lax.fori_loop(0, n_steps // _N_ACC, group, accs, unroll=3)

        fire(base, 0)

        def pair_body(ip, accs):
            for k in (0, 1):
                i = 2 * ip + k
                b = base + i

                @pl.when(i + 1 < bpw)
                def _():
                    fire(b + 1, 1 - k)

                drain(b, k)
                accs = compute(k, accs)
            return accs

        zero = jnp.zeros((_L,), jnp.float32)
        accs0 = tuple((zero, zero) for _ in range(_N_ACC))
        accs = lax.fori_loop(0, bpw // 2, pair_body, accs0)
        lacc = (accs[0][0] + accs[1][0]) + (accs[2][0] + accs[3][0])
        cacc = (accs[0][1] + accs[1][1]) + (accs[2][1] + accs[3][1])
        loss_v[...] = lacc
        cnt_v[...] = cacc
        pltpu.sync_copy(loss_v, loss_out.at[wid])
        pltpu.sync_copy(cnt_v, cnt_out.at[wid])

    return sc_kernel


def _finish(loss_ref, cnt_ref, out_ref):
    ls = jnp.sum(loss_ref[...])
    nv = jnp.sum(cnt_ref[...])
    out_ref[...] = (ls / (nv + 1e-6)).reshape(1, 1)


def kernel(curr_heightmap, new_roi, mask):
    B, _, H, W = curr_heightmap.shape
    HW = H * W
    curr2 = curr_heightmap.reshape(B, HW)
    mask2 = mask.reshape(B, HW)
    roi3 = jnp.transpose(new_roi, (1, 0, 2, 3)).reshape(3, B, HW)
    loss_p, cnt_p = _make_sc_partials(B, H, W)(curr2, mask2, roi3)
    out = pl.pallas_call(
        _finish,
        out_shape=jax.ShapeDtypeStruct((1, 1), jnp.float32),
    )(loss_p, cnt_p)
    return out[0, 0]


# speedup vs baseline: 1.0544x; 1.0544x over previous
import functools

import jax
import jax.numpy as jnp
from jax import lax
from jax.experimental import pallas as pl
from jax.experimental.pallas import tpu as pltpu
from jax.experimental.pallas import tpu_sc as plsc

_NC = 2
_NS = 16
_NW = _NC * _NS
_L = 16
_N_ACC = 4


def _make_sc_partials(B, H, W):
    HW = H * W
    assert B % (2 * _NW) == 0 and HW % (_L * _N_ACC) == 0
    bpw = B // _NW
    n_steps = HW // _L
    mesh = plsc.VectorSubcoreMesh(core_axis_name="c", subcore_axis_name="s")

    @functools.partial(
        pl.kernel,
        mesh=mesh,
        compiler_params=pltpu.CompilerParams(needs_layout_passes=False),
        out_type=[
            jax.ShapeDtypeStruct((_NW, _L), jnp.float32),
            jax.ShapeDtypeStruct((_NW, _L), jnp.float32),
        ],
        scratch_types=[
            pltpu.VMEM((HW,), jnp.float32),
            pltpu.VMEM((HW,), jnp.float32),
            pltpu.VMEM((HW,), jnp.float32),
            pltpu.VMEM((HW,), jnp.float32),
            pltpu.VMEM((HW,), jnp.float32),
            pltpu.VMEM((HW,), jnp.float32),
            pltpu.VMEM((HW,), jnp.float32),
            pltpu.VMEM((HW,), jnp.float32),
            pltpu.VMEM((HW,), jnp.float32),
            pltpu.VMEM((HW,), jnp.float32),
            pltpu.VMEM((_L,), jnp.float32),
            pltpu.VMEM((_L,), jnp.float32),
            pltpu.SemaphoreType.DMA,
            pltpu.SemaphoreType.DMA,
        ],
    )
    def sc_kernel(curr_hbm, mask_hbm, roi_hbm, loss_out, cnt_out,
                  y_v0, y_v1, x_v0, x_v1, t_v0, t_v1,
                  curr_v0, curr_v1, mask_v0, mask_v1,
                  loss_v, cnt_v, sem0, sem1):
        wid = lax.axis_index("s") * _NC + lax.axis_index("c")
        base = wid * bpw
        bufs = ((y_v0, x_v0, t_v0, curr_v0, mask_v0, sem0),
                (y_v1, x_v1, t_v1, curr_v1, mask_v1, sem1))

        def fire(b, k):
            y_v, x_v, t_v, curr_v, mask_v, sem = bufs[k]
            pltpu.make_async_copy(curr_hbm.at[b], curr_v, sem).start()
            pltpu.make_async_copy(mask_hbm.at[b], mask_v, sem).start()
            pltpu.make_async_copy(roi_hbm.at[0, b], y_v, sem).start()
            pltpu.make_async_copy(roi_hbm.at[1, b], x_v, sem).start()
            pltpu.make_async_copy(roi_hbm.at[2, b], t_v, sem).start()

        def drain(b, k):
            y_v, x_v, t_v, curr_v, mask_v, sem = bufs[k]
            pltpu.make_async_copy(curr_hbm.at[b], curr_v, sem).wait()
            pltpu.make_async_copy(mask_hbm.at[b], mask_v, sem).wait()
            pltpu.make_async_copy(roi_hbm.at[0, b], y_v, sem).wait()
            pltpu.make_async_copy(roi_hbm.at[1, b], x_v, sem).wait()
            pltpu.make_async_copy(roi_hbm.at[2, b], t_v, sem).wait()

        def compute(k, accs):
            y_v, x_v, t_v, curr_v, mask_v, _ = bufs[k]

            def one(j):
                sl = pl.ds(j * _L, _L)
                y = y_v[sl].astype(jnp.int32)
                x = x_v[sl].astype(jnp.int32)
                t = t_v[sl]
                valid = (lax.bitcast_convert_type(y, jnp.uint32) < H) & (
                    lax.bitcast_convert_type(x, jnp.uint32) < W)
                flat_u = lax.bitcast_convert_type(y * W + x, jnp.uint32)
                flat = lax.bitcast_convert_type(
                    jnp.minimum(flat_u, jnp.uint32(HW - 1)), jnp.int32)
                c = plsc.load_gather(curr_v, [flat])
                m = plsc.load_gather(mask_v, [flat])
                d = c - t
                ad = jnp.abs(d)
                loss = jnp.where(ad < 1.0, 0.5 * d * d, ad - 0.5)
                w = jnp.where(valid, m, 0.0)
                return loss * w, w

            def group(g, accs2):
                out = []
                for q in range(_N_ACC):
                    lacc, cacc = accs2[q]
                    lw, w = one(g * _N_ACC + q)
                    out.append((lacc + lw, cacc + w))
                return tuple(out)

            return lax.fori_loop(0, n_steps // _N_ACC, group, accs, unroll=2)

        fire(base, 0)

        def pair_body(ip, accs):
            for k in (0, 1):
                i = 2 * ip + k
                b = base + i

                @pl.when(i + 1 < bpw)
                def _():
                    fire(b + 1, 1 - k)

                drain(b, k)
                accs = compute(k, accs)
            return accs

        zero = jnp.zeros((_L,), jnp.float32)
        accs0 = tuple((zero, zero) for _ in range(_N_ACC))
        accs = lax.fori_loop(0, bpw // 2, pair_body, accs0)
        lacc = (accs[0][0] + accs[1][0]) + (accs[2][0] + accs[3][0])
        cacc = (accs[0][1] + accs[1][1]) + (accs[2][1] + accs[3][1])
        loss_v[...] = lacc
        cnt_v[...] = cacc
        pltpu.sync_copy(loss_v, loss_out.at[wid])
        pltpu.sync_copy(cnt_v, cnt_out.at[wid])

    return sc_kernel


def _finish(loss_ref, cnt_ref, out_ref):
    ls = jnp.sum(loss_ref[...])
    nv = jnp.sum(cnt_ref[...])
    out_ref[...] = (ls / (nv + 1e-6)).reshape(1, 1)


def kernel(curr_heightmap, new_roi, mask):
    B, _, H, W = curr_heightmap.shape
    HW = H * W
    curr2 = curr_heightmap.reshape(B, HW)
    mask2 = mask.reshape(B, HW)
    roi3 = jnp.transpose(new_roi, (1, 0, 2, 3)).reshape(3, B, HW)
    loss_p, cnt_p = _make_sc_partials(B, H, W)(curr2, mask2, roi3)
    out = pl.pallas_call(
        _finish,
        out_shape=jax.ShapeDtypeStruct((1, 1), jnp.float32),
    )(loss_p, cnt_p)
    return out[0, 0]
